# Initial kernel scaffold; baseline (speedup 1.0000x reference)
import jax
import jax.numpy as jnp
from jax.experimental import pallas as pl


def kernel(x, edge_index, W_l0, b_l0, W_r0, W_l1, b_l1, W_r1):
    raise NotImplementedError("write your pallas kernel here")


# speedup vs baseline: 4.1822x; 4.1822x over previous
import functools

import jax
import jax.numpy as jnp
from jax import lax
from jax.experimental import pallas as pl
from jax.experimental.pallas import tpu as pltpu
from jax.experimental.pallas import tpu_sc as plsc

N = 10000
D = 128
E = 320000

NC = 2
NS = 16
NW = NC * NS

K = 128
C_PT = 80
E_PT = C_PT * K
E_PAD = NW * E_PT

N_PAD = 10112
CNT_ROWS = 80
R_BLK = N_PAD // 16


def _zero_vmem_2d(ref, rows):
    z = jnp.zeros((16,), jnp.float32)

    def body(r, _):
        for k in range(8):
            ref[r, pl.ds(k * 16, 16)] = z
        return 0

    lax.fori_loop(0, rows, body, 0)


N_CNT = CNT_ROWS * 128


def _zero_vmem_1d(ref, n):
    z = jnp.zeros((16,), jnp.float32)

    def body(i, _):
        ref[pl.ds(i * 16, 16)] = z
        return 0

    lax.fori_loop(0, n // 16, body, 0)


def _cnt_body(dst_hbm, inv_hbm, dstbuf, cnt_v, acc, tmp, parts_sh):
    c = lax.axis_index("c")
    s = lax.axis_index("s")

    @pl.when(c == 0)
    def _():
        _zero_vmem_1d(cnt_v, N_CNT)

        ones = jnp.ones((16,), jnp.float32)
        e_pt = E_PAD // NS
        base = s * e_pt

        def chunk(ch, _):
            pltpu.sync_copy(dst_hbm.at[pl.ds(base + ch * 2048, 2048)], dstbuf)

            def grp(g, _):
                v = dstbuf[pl.ds(g * 16, 16)]
                plsc.addupdate_scatter(cnt_v, [v], ones)
                return 0

            lax.fori_loop(0, 128, grp, 0)
            return 0

        lax.fori_loop(0, e_pt // 2048, chunk, 0)

        pltpu.sync_copy(cnt_v, parts_sh.at[s])
        plsc.subcore_barrier()

        @pl.when(s < 10)
        def _():
            off = s * 1024
            pltpu.sync_copy(parts_sh.at[0, pl.ds(off, 1024)], acc)
            for t in range(1, NS):
                pltpu.sync_copy(parts_sh.at[t, pl.ds(off, 1024)], tmp)

                def add(g, _):
                    sl = pl.ds(g * 16, 16)
                    acc[sl] = acc[sl] + tmp[sl]
                    return 0

                lax.fori_loop(0, 64, add, 0)

            def inv_g(g, _):
                sl = pl.ds(g * 16, 16)
                acc[sl] = 1.0 / jnp.maximum(acc[sl], 1.0)
                return 0

            lax.fori_loop(0, 64, inv_g, 0)
            pltpu.sync_copy(acc, inv_hbm.at[pl.ds(off, 1024)])


def _make_cnt_kernel():
    mesh = plsc.VectorSubcoreMesh(core_axis_name="c", subcore_axis_name="s")
    return pl.kernel(
        _cnt_body,
        out_type=jax.ShapeDtypeStruct((N_CNT,), jnp.float32),
        mesh=mesh,
        compiler_params=pltpu.CompilerParams(needs_layout_passes=False),
        scratch_types=[
            pltpu.VMEM((2048,), jnp.int32),
            pltpu.VMEM((N_CNT,), jnp.float32),
            pltpu.VMEM((1024,), jnp.float32),
            pltpu.VMEM((1024,), jnp.float32),
            pltpu.VMEM_SHARED((NS, N_CNT), jnp.float32),
        ],
    )


W_PT = E_PT // 2


def _unpack_idx(packed, j, stage):
    for g in range(4):
        w = packed[pl.ds(j * 64 + g * 16, 16)]
        stage[0, pl.ds(g * 16, 16)] = lax.bitwise_and(w, 0xFFFF)
        stage[0, pl.ds(64 + g * 16, 16)] = lax.shift_right_logical(w, 16)


def _agg_body(y_hbm, src_hbm, dst_hbm, out_hbm,
              src_v, dst_v, st_sa, st_sb, st_d, buf_a, buf_b, agg_sh,
              sem_a, sem_b):
    c = lax.axis_index("c")
    s = lax.axis_index("s")
    wid = c * NS + s

    pltpu.sync_copy(src_hbm.at[wid], src_v)
    pltpu.sync_copy(dst_hbm.at[wid], dst_v)

    _zero_vmem_2d(buf_a, K)
    base = s * R_BLK
    for off in range(0, 512, 128):
        pltpu.sync_copy(buf_a, agg_sh.at[pl.ds(base + off, 128)])
    pltpu.sync_copy(buf_a.at[pl.ds(0, R_BLK - 512)],
                    agg_sh.at[pl.ds(base + 512, R_BLK - 512)])
    plsc.subcore_barrier()

    def gather(j, buf, stage, sem):
        _unpack_idx(src_v, j, stage)
        pltpu.async_copy(y_hbm.at[stage.at[0]], buf, sem)

    def wait_g(buf, sem):
        pltpu.make_async_copy(y_hbm.at[pl.ds(0, K)], buf, sem).wait()

    def scat(j, buf):
        _unpack_idx(dst_v, j, st_d)
        pltpu.sync_copy(buf, agg_sh.at[st_d.at[0]], add=True)

    gather(0, buf_a, st_sa, sem_a)

    def pair(jj, _):
        j0 = 2 * jj
        wait_g(buf_a, sem_a)
        gather(j0 + 1, buf_b, st_sb, sem_b)
        scat(j0, buf_a)
        wait_g(buf_b, sem_b)
        gather(j0 + 2, buf_a, st_sa, sem_a)
        scat(j0 + 1, buf_b)
        return 0

    lax.fori_loop(0, C_PT // 2 - 1, pair, 0)

    wait_g(buf_a, sem_a)
    gather(C_PT - 1, buf_b, st_sb, sem_b)
    scat(C_PT - 2, buf_a)
    wait_g(buf_b, sem_b)
    scat(C_PT - 1, buf_b)

    plsc.subcore_barrier()
    pltpu.sync_copy(agg_sh.at[pl.ds(base, R_BLK)],
                    out_hbm.at[c, pl.ds(base, R_BLK)])


def _make_agg_kernel():
    mesh = plsc.VectorSubcoreMesh(core_axis_name="c", subcore_axis_name="s")
    return pl.kernel(
        _agg_body,
        out_type=jax.ShapeDtypeStruct((NC, N_PAD, 128), jnp.float32),
        mesh=mesh,
        compiler_params=pltpu.CompilerParams(needs_layout_passes=False),
        scratch_types=[
            pltpu.VMEM((W_PT,), jnp.int32),
            pltpu.VMEM((W_PT,), jnp.int32),
            pltpu.VMEM((1, K), jnp.int32),
            pltpu.VMEM((1, K), jnp.int32),
            pltpu.VMEM((1, K), jnp.int32),
            pltpu.VMEM((K, 128), jnp.float32),
            pltpu.VMEM((K, 128), jnp.float32),
            pltpu.VMEM_SHARED((N_PAD, 128), jnp.float32),
            pltpu.SemaphoreType.DMA,
            pltpu.SemaphoreType.DMA,
        ],
    )


def _lin0_body(x_ref, wl_ref, wr_ref, b_ref, y_ref, z_ref):
    xb = x_ref[...]
    y_ref[...] = jnp.dot(xb, wl_ref[...], preferred_element_type=jnp.float32)
    z_ref[...] = (jnp.dot(xb, wr_ref[...], preferred_element_type=jnp.float32)
                  + b_ref[...])


def _mid_body(a0_ref, a1_ref, inv_ref, z0_ref, wl_ref, wr_ref, b_ref,
              y_ref, z_ref):
    h = (a0_ref[...] + a1_ref[...]) * inv_ref[...] + z0_ref[...]
    h = jnp.maximum(h, 0.0)
    y_ref[...] = jnp.dot(h, wl_ref[...], preferred_element_type=jnp.float32)
    z_ref[...] = (jnp.dot(h, wr_ref[...], preferred_element_type=jnp.float32)
                  + b_ref[...])


def _fin_body(a0_ref, a1_ref, inv_ref, z_ref, o_ref):
    o_ref[...] = (a0_ref[...] + a1_ref[...]) * inv_ref[...] + z_ref[...]


def _row_spec():
    return pl.BlockSpec((R_BLK, 128), lambda i: (i, 0))


def _col_spec():
    return pl.BlockSpec((R_BLK, 1), lambda i: (i, 0))


def _full_spec(shape):
    return pl.BlockSpec(shape, lambda i: tuple(0 for _ in shape))


def _lin0(x_pad, W_l, W_r, b):
    return pl.pallas_call(
        _lin0_body,
        grid=(16,),
        in_specs=[_row_spec(), _full_spec((128, 128)), _full_spec((128, 128)),
                  _full_spec((1, 128))],
        out_specs=[_row_spec(), _row_spec()],
        out_shape=[jax.ShapeDtypeStruct((N_PAD, 128), jnp.float32),
                   jax.ShapeDtypeStruct((N_PAD, 128), jnp.float32)],
    )(x_pad, W_l, W_r, b)


def _mid(a0, a1, inv_col, z0, W_l, W_r, b):
    return pl.pallas_call(
        _mid_body,
        grid=(16,),
        in_specs=[_row_spec(), _row_spec(), _col_spec(), _row_spec(),
                  _full_spec((128, 128)), _full_spec((128, 128)),
                  _full_spec((1, 128))],
        out_specs=[_row_spec(), _row_spec()],
        out_shape=[jax.ShapeDtypeStruct((N_PAD, 128), jnp.float32),
                   jax.ShapeDtypeStruct((N_PAD, 128), jnp.float32)],
    )(a0, a1, inv_col, z0, W_l, W_r, b)


def _fin(a0, a1, inv_col, z1):
    return pl.pallas_call(
        _fin_body,
        grid=(16,),
        in_specs=[_row_spec(), _row_spec(), _col_spec(), _row_spec()],
        out_specs=pl.BlockSpec((R_BLK, 128), lambda i: (i, 0)),
        out_shape=jax.ShapeDtypeStruct((N_PAD, 128), jnp.float32),
    )(a0, a1, inv_col, z1)


def kernel(x, edge_index, W_l0, b_l0, W_r0, W_l1, b_l1, W_r1):
    src = edge_index[0]
    dst = edge_index[1]

    pad_n = E_PAD - E
    pad_ar = jnp.arange(pad_n, dtype=jnp.int32)
    src_p = jnp.concatenate([src, pad_ar % N])
    dst_flat = jnp.concatenate([dst, N + pad_ar % (N_PAD - N)])
    src_p = src_p.reshape(NW, W_PT, 2)
    src_p = (src_p[..., 0] | (src_p[..., 1] << 16)).astype(jnp.int32)
    dst_p = dst_flat.reshape(NW, W_PT, 2)
    dst_p = (dst_p[..., 0] | (dst_p[..., 1] << 16)).astype(jnp.int32)

    x_pad = jnp.pad(x, ((0, N_PAD - N), (0, 0)))

    cnt_kernel = _make_cnt_kernel()
    agg_kernel = _make_agg_kernel()

    inv = cnt_kernel(dst_flat)
    inv_col = inv[:N_PAD].reshape(N_PAD, 1)

    y0, z0 = _lin0(x_pad, W_l0, W_r0, b_l0.reshape(1, 128))
    agg0 = agg_kernel(y0, src_p, dst_p)
    y1, z1 = _mid(agg0[0], agg0[1], inv_col, z0, W_l1, W_r1,
                  b_l1.reshape(1, 128))
    agg1 = agg_kernel(y1, src_p, dst_p)
    out = _fin(agg1[0], agg1[1], inv_col, z1)
    return out[:N]


# speedup vs baseline: 10.2369x; 2.4477x over previous
import functools

import jax
import jax.numpy as jnp
from jax import lax
from jax.experimental import pallas as pl
from jax.experimental.pallas import tpu as pltpu
from jax.experimental.pallas import tpu_sc as plsc

N = 10000
D = 128
E = 320000

NC = 2
NS = 16
NW = NC * NS

K = 128
C_PT = 80
E_PT = C_PT * K
E_PAD = NW * E_PT

N_PAD = 10112
CNT_ROWS = 80
R_BLK = N_PAD // 16


def _zero_vmem_2d(ref, rows):
    z = jnp.zeros((16,), jnp.float32)

    def body(r, _):
        for k in range(8):
            ref[r, pl.ds(k * 16, 16)] = z
        return 0

    lax.fori_loop(0, rows, body, 0)


N_CNT = CNT_ROWS * 128


def _zero_vmem_1d(ref, n):
    z = jnp.zeros((16,), jnp.float32)

    def body(i, _):
        ref[pl.ds(i * 16, 16)] = z
        return 0

    lax.fori_loop(0, n // 16, body, 0)


def _cnt_body(dst_hbm, inv_hbm, dstbuf, cnt_v, acc, tmp, parts_sh):
    c = lax.axis_index("c")
    s = lax.axis_index("s")

    @pl.when(c == 0)
    def _():
        _zero_vmem_1d(cnt_v, N_CNT)

        ones = jnp.ones((16,), jnp.float32)
        w_pt = (E_PAD // 2) // NS
        base = s * w_pt

        def chunk(ch, _):
            pltpu.sync_copy(dst_hbm.at[pl.ds(base + ch * 2048, 2048)], dstbuf)

            def grp(g, _):
                w = dstbuf[pl.ds(g * 16, 16)]
                plsc.addupdate_scatter(cnt_v, [lax.bitwise_and(w, 0xFFFF)],
                                       ones)
                plsc.addupdate_scatter(cnt_v,
                                       [lax.shift_right_logical(w, 16)], ones)
                return 0

            lax.fori_loop(0, 128, grp, 0)
            return 0

        lax.fori_loop(0, w_pt // 2048, chunk, 0)

        pltpu.sync_copy(cnt_v, parts_sh.at[s])
        plsc.subcore_barrier()

        @pl.when(s < 10)
        def _():
            off = s * 1024
            pltpu.sync_copy(parts_sh.at[0, pl.ds(off, 1024)], acc)
            for t in range(1, NS):
                pltpu.sync_copy(parts_sh.at[t, pl.ds(off, 1024)], tmp)

                def add(g, _):
                    sl = pl.ds(g * 16, 16)
                    acc[sl] = acc[sl] + tmp[sl]
                    return 0

                lax.fori_loop(0, 64, add, 0)

            def inv_g(g, _):
                sl = pl.ds(g * 16, 16)
                acc[sl] = 1.0 / jnp.maximum(acc[sl], 1.0)
                return 0

            lax.fori_loop(0, 64, inv_g, 0)
            pltpu.sync_copy(acc, inv_hbm.at[pl.ds(off, 1024)])


def _make_cnt_kernel():
    mesh = plsc.VectorSubcoreMesh(core_axis_name="c", subcore_axis_name="s")
    return pl.kernel(
        _cnt_body,
        out_type=jax.ShapeDtypeStruct((N_CNT,), jnp.float32),
        mesh=mesh,
        compiler_params=pltpu.CompilerParams(needs_layout_passes=False),
        scratch_types=[
            pltpu.VMEM((2048,), jnp.int32),
            pltpu.VMEM((N_CNT,), jnp.float32),
            pltpu.VMEM((1024,), jnp.float32),
            pltpu.VMEM((1024,), jnp.float32),
            pltpu.VMEM_SHARED((NS, N_CNT), jnp.float32),
        ],
    )


W_PT = E_PT // 2


def _unpack_idx(packed, j, stage):
    for g in range(4):
        w = packed[pl.ds(j * 64 + g * 16, 16)]
        stage[0, pl.ds(g * 16, 16)] = lax.bitwise_and(w, 0xFFFF)
        stage[0, pl.ds(64 + g * 16, 16)] = lax.shift_right_logical(w, 16)


def _agg_body(y_hbm, src_hbm, dst_hbm, out_hbm,
              src_v, dst_v, st_sa, st_sb, st_d, buf_a, buf_b, agg_sh,
              sem_a, sem_b):
    c = lax.axis_index("c")
    s = lax.axis_index("s")
    wid = c * NS + s

    pltpu.sync_copy(src_hbm.at[wid], src_v)
    pltpu.sync_copy(dst_hbm.at[wid], dst_v)

    _zero_vmem_2d(buf_a, K)
    base = s * R_BLK
    for off in range(0, 512, 128):
        pltpu.sync_copy(buf_a, agg_sh.at[pl.ds(base + off, 128)])
    pltpu.sync_copy(buf_a.at[pl.ds(0, R_BLK - 512)],
                    agg_sh.at[pl.ds(base + 512, R_BLK - 512)])
    plsc.subcore_barrier()

    def gather(j, buf, stage, sem):
        _unpack_idx(src_v, j, stage)
        pltpu.async_copy(y_hbm.at[stage.at[0]], buf, sem)

    def wait_g(buf, sem):
        pltpu.make_async_copy(y_hbm.at[pl.ds(0, K)], buf, sem).wait()

    def scat(j, buf):
        _unpack_idx(dst_v, j, st_d)
        pltpu.sync_copy(buf, agg_sh.at[st_d.at[0]], add=True)

    gather(0, buf_a, st_sa, sem_a)

    def pair(jj, _):
        j0 = 2 * jj
        wait_g(buf_a, sem_a)
        gather(j0 + 1, buf_b, st_sb, sem_b)
        scat(j0, buf_a)
        wait_g(buf_b, sem_b)
        gather(j0 + 2, buf_a, st_sa, sem_a)
        scat(j0 + 1, buf_b)
        return 0

    lax.fori_loop(0, C_PT // 2 - 1, pair, 0)

    wait_g(buf_a, sem_a)
    gather(C_PT - 1, buf_b, st_sb, sem_b)
    scat(C_PT - 2, buf_a)
    wait_g(buf_b, sem_b)
    scat(C_PT - 1, buf_b)

    plsc.subcore_barrier()
    pltpu.sync_copy(agg_sh.at[pl.ds(base, R_BLK)],
                    out_hbm.at[c, pl.ds(base, R_BLK)])


def _make_agg_kernel():
    mesh = plsc.VectorSubcoreMesh(core_axis_name="c", subcore_axis_name="s")
    return pl.kernel(
        _agg_body,
        out_type=jax.ShapeDtypeStruct((NC, N_PAD, 128), jnp.float32),
        mesh=mesh,
        compiler_params=pltpu.CompilerParams(needs_layout_passes=False),
        scratch_types=[
            pltpu.VMEM((W_PT,), jnp.int32),
            pltpu.VMEM((W_PT,), jnp.int32),
            pltpu.VMEM((1, K), jnp.int32),
            pltpu.VMEM((1, K), jnp.int32),
            pltpu.VMEM((1, K), jnp.int32),
            pltpu.VMEM((K, 128), jnp.float32),
            pltpu.VMEM((K, 128), jnp.float32),
            pltpu.VMEM_SHARED((N_PAD, 128), jnp.float32),
            pltpu.SemaphoreType.DMA,
            pltpu.SemaphoreType.DMA,
        ],
    )


def _lin0_body(x_ref, wl_ref, wr_ref, b_ref, y_ref, z_ref):
    xb = x_ref[...]
    y_ref[...] = jnp.dot(xb, wl_ref[...], preferred_element_type=jnp.float32)
    z_ref[...] = (jnp.dot(xb, wr_ref[...], preferred_element_type=jnp.float32)
                  + b_ref[...])


def _mid_body(a0_ref, a1_ref, inv_ref, z0_ref, wl_ref, wr_ref, b_ref,
              y_ref, z_ref):
    h = (a0_ref[...] + a1_ref[...]) * inv_ref[...] + z0_ref[...]
    h = jnp.maximum(h, 0.0)
    y_ref[...] = jnp.dot(h, wl_ref[...], preferred_element_type=jnp.float32)
    z_ref[...] = (jnp.dot(h, wr_ref[...], preferred_element_type=jnp.float32)
                  + b_ref[...])


def _fin_body(a0_ref, a1_ref, inv_ref, z_ref, o_ref):
    o_ref[...] = (a0_ref[...] + a1_ref[...]) * inv_ref[...] + z_ref[...]


def _row_spec():
    return pl.BlockSpec((R_BLK, 128), lambda i: (i, 0))


def _col_spec():
    return pl.BlockSpec((R_BLK, 1), lambda i: (i, 0))


def _full_spec(shape):
    return pl.BlockSpec(shape, lambda i: tuple(0 for _ in shape))


def _lin0(x_pad, W_l, W_r, b):
    return pl.pallas_call(
        _lin0_body,
        grid=(16,),
        in_specs=[_row_spec(), _full_spec((128, 128)), _full_spec((128, 128)),
                  _full_spec((1, 128))],
        out_specs=[_row_spec(), _row_spec()],
        out_shape=[jax.ShapeDtypeStruct((N_PAD, 128), jnp.float32),
                   jax.ShapeDtypeStruct((N_PAD, 128), jnp.float32)],
    )(x_pad, W_l, W_r, b)


def _mid(a0, a1, inv_col, z0, W_l, W_r, b):
    return pl.pallas_call(
        _mid_body,
        grid=(16,),
        in_specs=[_row_spec(), _row_spec(), _col_spec(), _row_spec(),
                  _full_spec((128, 128)), _full_spec((128, 128)),
                  _full_spec((1, 128))],
        out_specs=[_row_spec(), _row_spec()],
        out_shape=[jax.ShapeDtypeStruct((N_PAD, 128), jnp.float32),
                   jax.ShapeDtypeStruct((N_PAD, 128), jnp.float32)],
    )(a0, a1, inv_col, z0, W_l, W_r, b)


def _fin(a0, a1, inv_col, z1):
    return pl.pallas_call(
        _fin_body,
        grid=(16,),
        in_specs=[_row_spec(), _row_spec(), _col_spec(), _row_spec()],
        out_specs=pl.BlockSpec((R_BLK, 128), lambda i: (i, 0)),
        out_shape=jax.ShapeDtypeStruct((N_PAD, 128), jnp.float32),
    )(a0, a1, inv_col, z1)


def kernel(x, edge_index, W_l0, b_l0, W_r0, W_l1, b_l1, W_r1):
    src = edge_index[0]
    dst = edge_index[1]

    pad_n = E_PAD - E
    pad_ar = jnp.arange(pad_n, dtype=jnp.int32)
    src_f = jnp.concatenate([src, pad_ar % N])
    dst_f = jnp.concatenate([dst, N + pad_ar % (N_PAD - N)])
    half = E_PAD // 2
    src_p = (src_f[:half] | (src_f[half:] << 16)).reshape(NW, W_PT)
    dst_p = (dst_f[:half] | (dst_f[half:] << 16)).reshape(NW, W_PT)
    dst_flat = dst_p.reshape(-1)

    x_pad = jnp.pad(x, ((0, N_PAD - N), (0, 0)))

    cnt_kernel = _make_cnt_kernel()
    agg_kernel = _make_agg_kernel()

    inv = cnt_kernel(dst_flat)
    inv_col = inv[:N_PAD].reshape(N_PAD, 1)

    y0, z0 = _lin0(x_pad, W_l0, W_r0, b_l0.reshape(1, 128))
    agg0 = agg_kernel(y0, src_p, dst_p)
    y1, z1 = _mid(agg0[0], agg0[1], inv_col, z0, W_l1, W_r1,
                  b_l1.reshape(1, 128))
    agg1 = agg_kernel(y1, src_p, dst_p)
    out = _fin(agg1[0], agg1[1], inv_col, z1)
    return out[:N]


# speedup vs baseline: 10.7694x; 1.0520x over previous
import functools

import jax
import jax.numpy as jnp
from jax import lax
from jax.experimental import pallas as pl
from jax.experimental.pallas import tpu as pltpu
from jax.experimental.pallas import tpu_sc as plsc

N = 10000
D = 128
E = 320000

NC = 2
NS = 16
NW = NC * NS

K = 64
C_PT = 160
E_PT = C_PT * K
E_PAD = NW * E_PT

N_PAD = 10112
CNT_ROWS = 80
R_BLK = N_PAD // 16


def _zero_vmem_2d(ref, rows):
    z = jnp.zeros((16,), jnp.float32)

    def body(r, _):
        for k in range(8):
            ref[r, pl.ds(k * 16, 16)] = z
        return 0

    lax.fori_loop(0, rows, body, 0)


N_CNT = CNT_ROWS * 128


def _zero_vmem_1d(ref, n):
    z = jnp.zeros((16,), jnp.float32)

    def body(i, _):
        ref[pl.ds(i * 16, 16)] = z
        return 0

    lax.fori_loop(0, n // 16, body, 0)


def _cnt_body(dst_hbm, inv_hbm, dstbuf, cnt_v, acc, tmp, parts_sh):
    c = lax.axis_index("c")
    s = lax.axis_index("s")

    @pl.when(c == 0)
    def _():
        _zero_vmem_1d(cnt_v, N_CNT)

        ones = jnp.ones((16,), jnp.float32)
        w_pt = (E_PAD // 2) // NS
        base = s * w_pt

        def chunk(ch, _):
            pltpu.sync_copy(dst_hbm.at[pl.ds(base + ch * 2048, 2048)], dstbuf)

            def grp(g, _):
                w = dstbuf[pl.ds(g * 16, 16)]
                plsc.addupdate_scatter(cnt_v, [lax.bitwise_and(w, 0xFFFF)],
                                       ones)
                plsc.addupdate_scatter(cnt_v,
                                       [lax.shift_right_logical(w, 16)], ones)
                return 0

            lax.fori_loop(0, 128, grp, 0)
            return 0

        lax.fori_loop(0, w_pt // 2048, chunk, 0)

        pltpu.sync_copy(cnt_v, parts_sh.at[s])
        plsc.subcore_barrier()

        @pl.when(s < 10)
        def _():
            off = s * 1024
            pltpu.sync_copy(parts_sh.at[0, pl.ds(off, 1024)], acc)
            for t in range(1, NS):
                pltpu.sync_copy(parts_sh.at[t, pl.ds(off, 1024)], tmp)

                def add(g, _):
                    sl = pl.ds(g * 16, 16)
                    acc[sl] = acc[sl] + tmp[sl]
                    return 0

                lax.fori_loop(0, 64, add, 0)

            def inv_g(g, _):
                sl = pl.ds(g * 16, 16)
                acc[sl] = 1.0 / jnp.maximum(acc[sl], 1.0)
                return 0

            lax.fori_loop(0, 64, inv_g, 0)
            pltpu.sync_copy(acc, inv_hbm.at[pl.ds(off, 1024)])


def _make_cnt_kernel():
    mesh = plsc.VectorSubcoreMesh(core_axis_name="c", subcore_axis_name="s")
    return pl.kernel(
        _cnt_body,
        out_type=jax.ShapeDtypeStruct((N_CNT,), jnp.float32),
        mesh=mesh,
        compiler_params=pltpu.CompilerParams(needs_layout_passes=False),
        scratch_types=[
            pltpu.VMEM((2048,), jnp.int32),
            pltpu.VMEM((N_CNT,), jnp.float32),
            pltpu.VMEM((1024,), jnp.float32),
            pltpu.VMEM((1024,), jnp.float32),
            pltpu.VMEM_SHARED((NS, N_CNT), jnp.float32),
        ],
    )


W_PT = E_PT // 2
NBUF = 4


def _unpack_idx(packed, j, stage):
    for g in range(K // 32):
        w = packed[pl.ds(j * (K // 2) + g * 16, 16)]
        stage[0, pl.ds(g * 16, 16)] = lax.bitwise_and(w, 0xFFFF)
        stage[0, pl.ds(K // 2 + g * 16, 16)] = lax.shift_right_logical(w, 16)


def _agg_body(y_hbm, src_hbm, dst_hbm, out_hbm, src_v, dst_v, *rest):
    st_s = rest[0:NBUF]
    st_d = rest[NBUF:2 * NBUF]
    bufs = rest[2 * NBUF:3 * NBUF]
    agg_sh = rest[3 * NBUF]
    sem_g = rest[3 * NBUF + 1:3 * NBUF + 1 + NBUF]
    sem_s = rest[3 * NBUF + 1 + NBUF:]

    c = lax.axis_index("c")
    s = lax.axis_index("s")
    wid = c * NS + s

    pltpu.sync_copy(src_hbm.at[wid], src_v)
    pltpu.sync_copy(dst_hbm.at[wid], dst_v)

    _zero_vmem_2d(bufs[0], K)
    base = s * R_BLK
    nfull = (R_BLK // K) * K
    for off in range(0, nfull, K):
        pltpu.sync_copy(bufs[0], agg_sh.at[pl.ds(base + off, K)])
    if R_BLK > nfull:
        pltpu.sync_copy(bufs[0].at[pl.ds(0, R_BLK - nfull)],
                        agg_sh.at[pl.ds(base + nfull, R_BLK - nfull)])
    plsc.subcore_barrier()

    def gather_start(j, u):
        _unpack_idx(src_v, j, st_s[u])
        pltpu.async_copy(y_hbm.at[st_s[u].at[0]], bufs[u], sem_g[u])

    def wait_g(u):
        pltpu.make_async_copy(y_hbm.at[pl.ds(0, K)], bufs[u],
                              sem_g[u]).wait()

    def scat_start(j, u):
        _unpack_idx(dst_v, j, st_d[u])
        pltpu.async_copy(bufs[u], agg_sh.at[st_d[u].at[0]], sem_s[u],
                         add=True)

    def wait_s(u):
        pltpu.make_async_copy(bufs[u], agg_sh.at[pl.ds(0, K)],
                              sem_s[u]).wait()

    def slot(j, u, do_wait_s, gather_ahead):
        wait_g(u)
        scat_start(j, u)
        if gather_ahead:
            u2 = (u + 2) % NBUF
            if do_wait_s:
                wait_s(u2)
            gather_start(j + 2, u2)

    gather_start(0, 0)
    gather_start(1, 1)
    slot(0, 0, False, True)
    slot(1, 1, False, True)

    def steady(i, _):
        j = 2 + 4 * i
        slot(j, 2, True, True)
        slot(j + 1, 3, True, True)
        slot(j + 2, 0, True, True)
        slot(j + 3, 1, True, True)
        return 0

    lax.fori_loop(0, (C_PT - 8) // 4, steady, 0)

    j = C_PT - 6
    slot(j, 2, True, True)
    slot(j + 1, 3, True, True)
    slot(j + 2, 0, True, True)
    slot(j + 3, 1, True, True)
    slot(j + 4, 2, False, False)
    slot(j + 5, 3, False, False)
    for u in range(NBUF):
        wait_s(u)

    plsc.subcore_barrier()
    pltpu.sync_copy(agg_sh.at[pl.ds(base, R_BLK)],
                    out_hbm.at[c, pl.ds(base, R_BLK)])


def _make_agg_kernel():
    mesh = plsc.VectorSubcoreMesh(core_axis_name="c", subcore_axis_name="s")
    return pl.kernel(
        _agg_body,
        out_type=jax.ShapeDtypeStruct((NC, N_PAD, 128), jnp.float32),
        mesh=mesh,
        compiler_params=pltpu.CompilerParams(needs_layout_passes=False),
        scratch_types=(
            [pltpu.VMEM((W_PT,), jnp.int32)] * 2
            + [pltpu.VMEM((1, K), jnp.int32)] * (2 * NBUF)
            + [pltpu.VMEM((K, 128), jnp.float32)] * NBUF
            + [pltpu.VMEM_SHARED((N_PAD, 128), jnp.float32)]
            + [pltpu.SemaphoreType.DMA] * (2 * NBUF)
        ),
    )


def _lin0_body(x_ref, wl_ref, wr_ref, b_ref, y_ref, z_ref):
    xb = x_ref[...]
    y_ref[...] = jnp.dot(xb, wl_ref[...], preferred_element_type=jnp.float32)
    z_ref[...] = (jnp.dot(xb, wr_ref[...], preferred_element_type=jnp.float32)
                  + b_ref[...])


def _mid_body(a0_ref, a1_ref, inv_ref, z0_ref, wl_ref, wr_ref, b_ref,
              y_ref, z_ref):
    h = (a0_ref[...] + a1_ref[...]) * inv_ref[...] + z0_ref[...]
    h = jnp.maximum(h, 0.0)
    y_ref[...] = jnp.dot(h, wl_ref[...], preferred_element_type=jnp.float32)
    z_ref[...] = (jnp.dot(h, wr_ref[...], preferred_element_type=jnp.float32)
                  + b_ref[...])


def _fin_body(a0_ref, a1_ref, inv_ref, z_ref, o_ref):
    o_ref[...] = (a0_ref[...] + a1_ref[...]) * inv_ref[...] + z_ref[...]


def _row_spec():
    return pl.BlockSpec((R_BLK, 128), lambda i: (i, 0))


def _col_spec():
    return pl.BlockSpec((R_BLK, 1), lambda i: (i, 0))


def _full_spec(shape):
    return pl.BlockSpec(shape, lambda i: tuple(0 for _ in shape))


def _lin0(x_pad, W_l, W_r, b):
    return pl.pallas_call(
        _lin0_body,
        grid=(16,),
        in_specs=[_row_spec(), _full_spec((128, 128)), _full_spec((128, 128)),
                  _full_spec((1, 128))],
        out_specs=[_row_spec(), _row_spec()],
        out_shape=[jax.ShapeDtypeStruct((N_PAD, 128), jnp.float32),
                   jax.ShapeDtypeStruct((N_PAD, 128), jnp.float32)],
    )(x_pad, W_l, W_r, b)


def _mid(a0, a1, inv_col, z0, W_l, W_r, b):
    return pl.pallas_call(
        _mid_body,
        grid=(16,),
        in_specs=[_row_spec(), _row_spec(), _col_spec(), _row_spec(),
                  _full_spec((128, 128)), _full_spec((128, 128)),
                  _full_spec((1, 128))],
        out_specs=[_row_spec(), _row_spec()],
        out_shape=[jax.ShapeDtypeStruct((N_PAD, 128), jnp.float32),
                   jax.ShapeDtypeStruct((N_PAD, 128), jnp.float32)],
    )(a0, a1, inv_col, z0, W_l, W_r, b)


def _fin(a0, a1, inv_col, z1):
    return pl.pallas_call(
        _fin_body,
        grid=(16,),
        in_specs=[_row_spec(), _row_spec(), _col_spec(), _row_spec()],
        out_specs=pl.BlockSpec((R_BLK, 128), lambda i: (i, 0)),
        out_shape=jax.ShapeDtypeStruct((N_PAD, 128), jnp.float32),
    )(a0, a1, inv_col, z1)


def kernel(x, edge_index, W_l0, b_l0, W_r0, W_l1, b_l1, W_r1):
    src = edge_index[0]
    dst = edge_index[1]

    pad_n = E_PAD - E
    pad_ar = jnp.arange(pad_n, dtype=jnp.int32)
    src_f = jnp.concatenate([src, pad_ar % N])
    dst_f = jnp.concatenate([dst, N + pad_ar % (N_PAD - N)])
    half = E_PAD // 2
    src_p = (src_f[:half] | (src_f[half:] << 16)).reshape(NW, W_PT)
    dst_p = (dst_f[:half] | (dst_f[half:] << 16)).reshape(NW, W_PT)
    dst_flat = dst_p.reshape(-1)

    x_pad = jnp.pad(x, ((0, N_PAD - N), (0, 0)))

    cnt_kernel = _make_cnt_kernel()
    agg_kernel = _make_agg_kernel()

    inv = cnt_kernel(dst_flat)
    inv_col = inv[:N_PAD].reshape(N_PAD, 1)

    y0, z0 = _lin0(x_pad, W_l0, W_r0, b_l0.reshape(1, 128))
    agg0 = agg_kernel(y0, src_p, dst_p)
    y1, z1 = _mid(agg0[0], agg0[1], inv_col, z0, W_l1, W_r1,
                  b_l1.reshape(1, 128))
    agg1 = agg_kernel(y1, src_p, dst_p)
    out = _fin(agg1[0], agg1[1], inv_col, z1)
    return out[:N]


# speedup vs baseline: 11.4616x; 1.0643x over previous
import functools

import jax
import jax.numpy as jnp
from jax import lax
from jax.experimental import pallas as pl
from jax.experimental.pallas import tpu as pltpu
from jax.experimental.pallas import tpu_sc as plsc

N = 10000
D = 128
E = 320000

NC = 2
NS = 16
NW = NC * NS

K = 64
C_PT = 160
E_PT = C_PT * K
E_PAD = NW * E_PT

N_PAD = 10112
CNT_ROWS = 80
R_BLK = N_PAD // 16


def _zero_vmem_2d(ref, rows):
    z = jnp.zeros((16,), jnp.float32)

    def body(r, _):
        for k in range(8):
            ref[r, pl.ds(k * 16, 16)] = z
        return 0

    lax.fori_loop(0, rows, body, 0)


N_CNT = CNT_ROWS * 128


def _zero_vmem_1d(ref, n):
    z = jnp.zeros((16,), jnp.float32)

    def body(i, _):
        ref[pl.ds(i * 16, 16)] = z
        return 0

    lax.fori_loop(0, n // 16, body, 0)


def _cnt_body(dst_hbm, inv_hbm, dstbuf, cnt_v, acc, tmp, parts_sh):
    c = lax.axis_index("c")
    s = lax.axis_index("s")

    @pl.when(c == 0)
    def _():
        _zero_vmem_1d(cnt_v, N_CNT)

        ones = jnp.ones((16,), jnp.float32)
        w_pt = (E_PAD // 2) // NS
        base = s * w_pt

        def chunk(ch, _):
            pltpu.sync_copy(dst_hbm.at[pl.ds(base + ch * 2048, 2048)], dstbuf)

            def grp(g, _):
                w = dstbuf[pl.ds(g * 16, 16)]
                plsc.addupdate_scatter(cnt_v, [lax.bitwise_and(w, 0xFFFF)],
                                       ones)
                plsc.addupdate_scatter(cnt_v,
                                       [lax.shift_right_logical(w, 16)], ones)
                return 0

            lax.fori_loop(0, 128, grp, 0)
            return 0

        lax.fori_loop(0, w_pt // 2048, chunk, 0)

        pltpu.sync_copy(cnt_v, parts_sh.at[s])
        plsc.subcore_barrier()

        @pl.when(s < 10)
        def _():
            off = s * 1024
            pltpu.sync_copy(parts_sh.at[0, pl.ds(off, 1024)], acc)
            for t in range(1, NS):
                pltpu.sync_copy(parts_sh.at[t, pl.ds(off, 1024)], tmp)

                def add(g, _):
                    sl = pl.ds(g * 16, 16)
                    acc[sl] = acc[sl] + tmp[sl]
                    return 0

                lax.fori_loop(0, 64, add, 0)

            def inv_g(g, _):
                sl = pl.ds(g * 16, 16)
                acc[sl] = 1.0 / jnp.maximum(acc[sl], 1.0)
                return 0

            lax.fori_loop(0, 64, inv_g, 0)
            pltpu.sync_copy(acc, inv_hbm.at[pl.ds(off, 1024)])


def _make_cnt_kernel():
    mesh = plsc.VectorSubcoreMesh(core_axis_name="c", subcore_axis_name="s")
    return pl.kernel(
        _cnt_body,
        out_type=jax.ShapeDtypeStruct((N_CNT,), jnp.float32),
        mesh=mesh,
        compiler_params=pltpu.CompilerParams(needs_layout_passes=False),
        scratch_types=[
            pltpu.VMEM((2048,), jnp.int32),
            pltpu.VMEM((N_CNT,), jnp.float32),
            pltpu.VMEM((1024,), jnp.float32),
            pltpu.VMEM((1024,), jnp.float32),
            pltpu.VMEM_SHARED((NS, N_CNT), jnp.float32),
        ],
    )


W_PT = E_PT // 2
NBUF = 4


def _unpack_idx(packed, j, stage):
    for g in range(K // 32):
        w = packed[pl.ds(j * (K // 2) + g * 16, 16)]
        stage[0, pl.ds(g * 16, 16)] = lax.bitwise_and(w, 0xFFFF)
        stage[0, pl.ds(K // 2 + g * 16, 16)] = lax.shift_right_logical(w, 16)


def _agg_body(y_hbm, src_hbm, dst_hbm, out_hbm, src_v, dst_v, *rest):
    st_s = rest[0:NBUF]
    st_d = rest[NBUF:2 * NBUF]
    bufs = rest[2 * NBUF:3 * NBUF]
    agg_sh = rest[3 * NBUF]
    sem_g = rest[3 * NBUF + 1:3 * NBUF + 1 + NBUF]
    sem_s = rest[3 * NBUF + 1 + NBUF:]

    c = lax.axis_index("c")
    s = lax.axis_index("s")
    wid = c * NS + s

    pltpu.sync_copy(src_hbm.at[wid], src_v)
    pltpu.sync_copy(dst_hbm.at[wid], dst_v)

    _zero_vmem_2d(bufs[0], K)
    base = s * R_BLK
    nfull = (R_BLK // K) * K
    for off in range(0, nfull, K):
        pltpu.sync_copy(bufs[0], agg_sh.at[pl.ds(base + off, K)])
    if R_BLK > nfull:
        pltpu.sync_copy(bufs[0].at[pl.ds(0, R_BLK - nfull)],
                        agg_sh.at[pl.ds(base + nfull, R_BLK - nfull)])
    plsc.subcore_barrier()

    def gather_start(j, u):
        _unpack_idx(src_v, j, st_s[u])
        pltpu.async_copy(y_hbm.at[st_s[u].at[0]], bufs[u], sem_g[u])

    def wait_g(u):
        pltpu.make_async_copy(y_hbm.at[pl.ds(0, K)], bufs[u],
                              sem_g[u]).wait()

    def scat_start(j, u):
        _unpack_idx(dst_v, j, st_d[u])
        pltpu.async_copy(bufs[u], agg_sh.at[st_d[u].at[0]], sem_s[u],
                         add=True)

    def wait_s(u):
        pltpu.make_async_copy(bufs[u], agg_sh.at[pl.ds(0, K)],
                              sem_s[u]).wait()

    def slot(j, u, do_wait_s, gather_ahead):
        wait_g(u)
        scat_start(j, u)
        if gather_ahead:
            u2 = (u + 2) % NBUF
            if do_wait_s:
                wait_s(u2)
            gather_start(j + 2, u2)

    gather_start(0, 0)
    gather_start(1, 1)
    slot(0, 0, False, True)
    slot(1, 1, False, True)

    def steady(i, _):
        j = 2 + 4 * i
        slot(j, 2, True, True)
        slot(j + 1, 3, True, True)
        slot(j + 2, 0, True, True)
        slot(j + 3, 1, True, True)
        return 0

    lax.fori_loop(0, (C_PT - 8) // 4, steady, 0)

    j = C_PT - 6
    slot(j, 2, True, True)
    slot(j + 1, 3, True, True)
    slot(j + 2, 0, True, True)
    slot(j + 3, 1, True, True)
    slot(j + 4, 2, False, False)
    slot(j + 5, 3, False, False)
    for u in range(NBUF):
        wait_s(u)

    plsc.subcore_barrier()
    pltpu.sync_copy(agg_sh.at[pl.ds(base, R_BLK)],
                    out_hbm.at[c, pl.ds(base, R_BLK)])


def _make_agg_kernel():
    mesh = plsc.VectorSubcoreMesh(core_axis_name="c", subcore_axis_name="s")
    return pl.kernel(
        _agg_body,
        out_type=jax.ShapeDtypeStruct((NC, N_PAD, 128), jnp.float32),
        mesh=mesh,
        compiler_params=pltpu.CompilerParams(needs_layout_passes=False),
        scratch_types=(
            [pltpu.VMEM((W_PT,), jnp.int32)] * 2
            + [pltpu.VMEM((1, K), jnp.int32)] * (2 * NBUF)
            + [pltpu.VMEM((K, 128), jnp.float32)] * NBUF
            + [pltpu.VMEM_SHARED((N_PAD, 128), jnp.float32)]
            + [pltpu.SemaphoreType.DMA] * (2 * NBUF)
        ),
    )


TC_BLK = 400
TC_GRID = N // TC_BLK
PACK_BL = 40960
PACK_GRID = (E_PAD // 2) // PACK_BL


def _pack_body(lo_ref, hi_ref, sp_ref, dp_ref):
    i = pl.program_id(0)
    w = ((i + PACK_GRID) * PACK_BL
         + jax.lax.broadcasted_iota(jnp.int32, (1, PACK_BL), 1))
    m = w < E
    pad_off = jnp.maximum(w - E, 0)
    hi_src = jnp.where(m, hi_ref[0:1], pad_off % N)
    hi_dst = jnp.where(m, hi_ref[1:2], N + pad_off % (N_PAD - N))
    sp_ref[0] = lo_ref[0:1] | (hi_src << 16)
    dp_ref[0] = lo_ref[1:2] | (hi_dst << 16)


def _pack(edge_index):
    sp, dp = pl.pallas_call(
        _pack_body,
        grid=(PACK_GRID,),
        in_specs=[pl.BlockSpec((2, PACK_BL), lambda i: (0, i)),
                  pl.BlockSpec((2, PACK_BL), lambda i: (0, i + PACK_GRID))],
        out_specs=[pl.BlockSpec((1, 1, PACK_BL), lambda i: (i, 0, 0)),
                   pl.BlockSpec((1, 1, PACK_BL), lambda i: (i, 0, 0))],
        out_shape=[jax.ShapeDtypeStruct((PACK_GRID, 1, PACK_BL), jnp.int32),
                   jax.ShapeDtypeStruct((PACK_GRID, 1, PACK_BL), jnp.int32)],
    )(edge_index, edge_index)
    return sp.reshape(NW, W_PT), dp.reshape(NW, W_PT)


def _lin0_body(x_ref, wl_ref, wr_ref, b_ref, y_ref, z_ref):
    xb = x_ref[...]
    y_ref[...] = jnp.dot(xb, wl_ref[...], preferred_element_type=jnp.float32)
    z_ref[...] = (jnp.dot(xb, wr_ref[...], preferred_element_type=jnp.float32)
                  + b_ref[...])


def _mid_body(a_ref, inv_ref, z0_ref, wl_ref, wr_ref, b_ref, y_ref, z_ref):
    h = (a_ref[0] + a_ref[1]) * inv_ref[...] + z0_ref[...]
    h = jnp.maximum(h, 0.0)
    y_ref[...] = jnp.dot(h, wl_ref[...], preferred_element_type=jnp.float32)
    z_ref[...] = (jnp.dot(h, wr_ref[...], preferred_element_type=jnp.float32)
                  + b_ref[...])


def _fin_body(a_ref, inv_ref, z_ref, o_ref):
    o_ref[...] = (a_ref[0] + a_ref[1]) * inv_ref[...] + z_ref[...]


def _row_spec():
    return pl.BlockSpec((TC_BLK, 128), lambda i: (i, 0))


def _agg_spec():
    return pl.BlockSpec((NC, TC_BLK, 128), lambda i: (0, i, 0))


def _col_spec():
    return pl.BlockSpec((TC_BLK, 1), lambda i: (i, 0))


def _full_spec(shape):
    return pl.BlockSpec(shape, lambda i: tuple(0 for _ in shape))


def _lin0(x, W_l, W_r, b):
    return pl.pallas_call(
        _lin0_body,
        grid=(TC_GRID,),
        in_specs=[_row_spec(), _full_spec((128, 128)), _full_spec((128, 128)),
                  _full_spec((1, 128))],
        out_specs=[_row_spec(), _row_spec()],
        out_shape=[jax.ShapeDtypeStruct((N, 128), jnp.float32),
                   jax.ShapeDtypeStruct((N, 128), jnp.float32)],
    )(x, W_l, W_r, b)


def _mid(agg, inv_col, z0, W_l, W_r, b):
    return pl.pallas_call(
        _mid_body,
        grid=(TC_GRID,),
        in_specs=[_agg_spec(), _col_spec(), _row_spec(),
                  _full_spec((128, 128)), _full_spec((128, 128)),
                  _full_spec((1, 128))],
        out_specs=[_row_spec(), _row_spec()],
        out_shape=[jax.ShapeDtypeStruct((N, 128), jnp.float32),
                   jax.ShapeDtypeStruct((N, 128), jnp.float32)],
    )(agg, inv_col, z0, W_l, W_r, b)


def _fin(agg, inv_col, z1):
    return pl.pallas_call(
        _fin_body,
        grid=(TC_GRID,),
        in_specs=[_agg_spec(), _col_spec(), _row_spec()],
        out_specs=pl.BlockSpec((TC_BLK, 128), lambda i: (i, 0)),
        out_shape=jax.ShapeDtypeStruct((N, 128), jnp.float32),
    )(agg, inv_col, z1)


def kernel(x, edge_index, W_l0, b_l0, W_r0, W_l1, b_l1, W_r1):
    src_p, dst_p = _pack(edge_index)
    dst_flat = dst_p.reshape(-1)

    cnt_kernel = _make_cnt_kernel()
    agg_kernel = _make_agg_kernel()

    inv = cnt_kernel(dst_flat)
    inv_col = inv[:N].reshape(N, 1)

    y0, z0 = _lin0(x, W_l0, W_r0, b_l0.reshape(1, 128))
    agg0 = agg_kernel(y0, src_p, dst_p)
    y1, z1 = _mid(agg0, inv_col, z0, W_l1, W_r1, b_l1.reshape(1, 128))
    agg1 = agg_kernel(y1, src_p, dst_p)
    return _fin(agg1, inv_col, z1)


# speedup vs baseline: 12.0802x; 1.0540x over previous
import functools

import jax
import jax.numpy as jnp
from jax import lax
from jax.experimental import pallas as pl
from jax.experimental.pallas import tpu as pltpu
from jax.experimental.pallas import tpu_sc as plsc

N = 10000
D = 128
E = 320000

NC = 2
NS = 16
NW = NC * NS

K = 64
C_PT = 160
E_PT = C_PT * K
E_PAD = NW * E_PT

N_PAD = 10112
CNT_ROWS = 80
R_BLK = N_PAD // 16


def _zero_vmem_2d(ref, rows):
    z = jnp.zeros((16,), jnp.float32)

    def body(r, _):
        for k in range(8):
            ref[r, pl.ds(k * 16, 16)] = z
        return 0

    lax.fori_loop(0, rows, body, 0)


N_CNT = CNT_ROWS * 128


def _zero_vmem_1d(ref, n):
    z = jnp.zeros((16,), jnp.float32)

    def body(i, _):
        ref[pl.ds(i * 16, 16)] = z
        return 0

    lax.fori_loop(0, n // 16, body, 0)


def _cnt_body(dst_hbm, inv_hbm, dstbuf, cnt_v, acc, tmp, parts_sh):
    c = lax.axis_index("c")
    s = lax.axis_index("s")

    @pl.when(c == 0)
    def _():
        _zero_vmem_1d(cnt_v, N_CNT)

        ones = jnp.ones((16,), jnp.float32)
        for r in range(2):
            def chunk(ch, _):
                pltpu.sync_copy(
                    dst_hbm.at[2 * s + r, pl.ds(ch * 1024, 1024)], dstbuf)

                def grp(g, _):
                    w = dstbuf[pl.ds(g * 16, 16)]
                    plsc.addupdate_scatter(
                        cnt_v, [lax.bitwise_and(w, 0xFFFF)], ones)
                    plsc.addupdate_scatter(
                        cnt_v, [lax.shift_right_logical(w, 16)], ones)
                    return 0

                lax.fori_loop(0, 64, grp, 0)
                return 0

            lax.fori_loop(0, W_PT // 1024, chunk, 0)

        pltpu.sync_copy(cnt_v, parts_sh.at[s])
        plsc.subcore_barrier()

        @pl.when(s < 10)
        def _():
            off = s * 1024
            pltpu.sync_copy(parts_sh.at[0, pl.ds(off, 1024)], acc)
            for t in range(1, NS):
                pltpu.sync_copy(parts_sh.at[t, pl.ds(off, 1024)], tmp)

                def add(g, _):
                    sl = pl.ds(g * 16, 16)
                    acc[sl] = acc[sl] + tmp[sl]
                    return 0

                lax.fori_loop(0, 64, add, 0)

            def inv_g(g, _):
                sl = pl.ds(g * 16, 16)
                acc[sl] = 1.0 / jnp.maximum(acc[sl], 1.0)
                return 0

            lax.fori_loop(0, 64, inv_g, 0)
            pltpu.sync_copy(acc, inv_hbm.at[pl.ds(off, 1024)])


def _make_cnt_kernel():
    mesh = plsc.VectorSubcoreMesh(core_axis_name="c", subcore_axis_name="s")
    return pl.kernel(
        _cnt_body,
        out_type=jax.ShapeDtypeStruct((N_CNT,), jnp.float32),
        mesh=mesh,
        compiler_params=pltpu.CompilerParams(needs_layout_passes=False),
        scratch_types=[
            pltpu.VMEM((1024,), jnp.int32),
            pltpu.VMEM((N_CNT,), jnp.float32),
            pltpu.VMEM((1024,), jnp.float32),
            pltpu.VMEM((1024,), jnp.float32),
            pltpu.VMEM_SHARED((NS, N_CNT), jnp.float32),
        ],
    )


W_PT = E_PT // 2
NBUF = 4


def _unpack_idx(packed, j, stage):
    for g in range(K // 32):
        w = packed[pl.ds(j * (K // 2) + g * 16, 16)]
        stage[0, pl.ds(g * 16, 16)] = lax.bitwise_and(w, 0xFFFF)
        stage[0, pl.ds(K // 2 + g * 16, 16)] = lax.shift_right_logical(w, 16)


def _agg_body(y_hbm, src_hbm, dst_hbm, out_hbm, src_v, dst_v, *rest):
    st_s = rest[0:NBUF]
    st_d = rest[NBUF:2 * NBUF]
    bufs = rest[2 * NBUF:3 * NBUF]
    agg_sh = rest[3 * NBUF]
    sem_g = rest[3 * NBUF + 1:3 * NBUF + 1 + NBUF]
    sem_s = rest[3 * NBUF + 1 + NBUF:]

    c = lax.axis_index("c")
    s = lax.axis_index("s")
    wid = c * NS + s

    pltpu.sync_copy(src_hbm.at[wid], src_v)
    pltpu.sync_copy(dst_hbm.at[wid], dst_v)

    _zero_vmem_2d(bufs[0], K)
    base = s * R_BLK
    nfull = (R_BLK // K) * K
    for off in range(0, nfull, K):
        pltpu.sync_copy(bufs[0], agg_sh.at[pl.ds(base + off, K)])
    if R_BLK > nfull:
        pltpu.sync_copy(bufs[0].at[pl.ds(0, R_BLK - nfull)],
                        agg_sh.at[pl.ds(base + nfull, R_BLK - nfull)])
    plsc.subcore_barrier()

    def gather_start(j, u):
        _unpack_idx(src_v, j, st_s[u])
        pltpu.async_copy(y_hbm.at[st_s[u].at[0]], bufs[u], sem_g[u])

    def wait_g(u):
        pltpu.make_async_copy(y_hbm.at[pl.ds(0, K)], bufs[u],
                              sem_g[u]).wait()

    def scat_start(j, u):
        _unpack_idx(dst_v, j, st_d[u])
        pltpu.async_copy(bufs[u], agg_sh.at[st_d[u].at[0]], sem_s[u],
                         add=True)

    def wait_s(u):
        pltpu.make_async_copy(bufs[u], agg_sh.at[pl.ds(0, K)],
                              sem_s[u]).wait()

    def slot(j, u, do_wait_s, gather_ahead):
        wait_g(u)
        scat_start(j, u)
        if gather_ahead:
            u2 = (u + 2) % NBUF
            if do_wait_s:
                wait_s(u2)
            gather_start(j + 2, u2)

    gather_start(0, 0)
    gather_start(1, 1)
    slot(0, 0, False, True)
    slot(1, 1, False, True)

    def steady(i, _):
        j = 2 + 4 * i
        slot(j, 2, True, True)
        slot(j + 1, 3, True, True)
        slot(j + 2, 0, True, True)
        slot(j + 3, 1, True, True)
        return 0

    lax.fori_loop(0, (C_PT - 8) // 4, steady, 0)

    j = C_PT - 6
    slot(j, 2, True, True)
    slot(j + 1, 3, True, True)
    slot(j + 2, 0, True, True)
    slot(j + 3, 1, True, True)
    slot(j + 4, 2, False, False)
    slot(j + 5, 3, False, False)
    for u in range(NBUF):
        wait_s(u)

    plsc.subcore_barrier()
    pltpu.sync_copy(agg_sh.at[pl.ds(base, R_BLK)],
                    out_hbm.at[c, pl.ds(base, R_BLK)])


def _make_agg_kernel():
    mesh = plsc.VectorSubcoreMesh(core_axis_name="c", subcore_axis_name="s")
    return pl.kernel(
        _agg_body,
        out_type=jax.ShapeDtypeStruct((NC, N_PAD, 128), jnp.float32),
        mesh=mesh,
        compiler_params=pltpu.CompilerParams(needs_layout_passes=False),
        scratch_types=(
            [pltpu.VMEM((W_PT,), jnp.int32)] * 2
            + [pltpu.VMEM((1, K), jnp.int32)] * (2 * NBUF)
            + [pltpu.VMEM((K, 128), jnp.float32)] * NBUF
            + [pltpu.VMEM_SHARED((N_PAD, 128), jnp.float32)]
            + [pltpu.SemaphoreType.DMA] * (2 * NBUF)
        ),
    )


TC_BLK = 2000
TC_GRID = N // TC_BLK
PACK_BL = 40960
PACK_GRID = (E_PAD // 2) // PACK_BL


def _pack_body(lo_ref, hi_ref, sp_ref, dp_ref):
    i = pl.program_id(0)
    w = ((i + PACK_GRID) * PACK_BL
         + jax.lax.broadcasted_iota(jnp.int32, (1, PACK_BL), 1))
    m = w < E
    pad_off = jnp.maximum(w - E, 0)
    hi_src = jnp.where(m, hi_ref[0:1], pad_off % N)
    hi_dst = jnp.where(m, hi_ref[1:2], N + pad_off % (N_PAD - N))
    sp_ref[...] = (lo_ref[0:1] | (hi_src << 16)).reshape(8, W_PT)
    dp_ref[...] = (lo_ref[1:2] | (hi_dst << 16)).reshape(8, W_PT)


def _pack(edge_index):
    sp, dp = pl.pallas_call(
        _pack_body,
        grid=(PACK_GRID,),
        in_specs=[pl.BlockSpec((2, PACK_BL), lambda i: (0, i)),
                  pl.BlockSpec((2, PACK_BL), lambda i: (0, i + PACK_GRID))],
        out_specs=[pl.BlockSpec((8, W_PT), lambda i: (i, 0)),
                   pl.BlockSpec((8, W_PT), lambda i: (i, 0))],
        out_shape=[jax.ShapeDtypeStruct((NW, W_PT), jnp.int32),
                   jax.ShapeDtypeStruct((NW, W_PT), jnp.int32)],
    )(edge_index, edge_index)
    return sp, dp


def _lin0_body(x_ref, wl_ref, wr_ref, b_ref, y_ref, z_ref):
    xb = x_ref[...]
    y_ref[...] = jnp.dot(xb, wl_ref[...], preferred_element_type=jnp.float32)
    z_ref[...] = (jnp.dot(xb, wr_ref[...], preferred_element_type=jnp.float32)
                  + b_ref[...])


def _mid_body(a_ref, inv_ref, z0_ref, wl_ref, wr_ref, b_ref, y_ref, z_ref):
    h = (a_ref[0] + a_ref[1]) * inv_ref[...] + z0_ref[...]
    h = jnp.maximum(h, 0.0)
    y_ref[...] = jnp.dot(h, wl_ref[...], preferred_element_type=jnp.float32)
    z_ref[...] = (jnp.dot(h, wr_ref[...], preferred_element_type=jnp.float32)
                  + b_ref[...])


def _fin_body(a_ref, inv_ref, z_ref, o_ref):
    o_ref[...] = (a_ref[0] + a_ref[1]) * inv_ref[...] + z_ref[...]


def _row_spec():
    return pl.BlockSpec((TC_BLK, 128), lambda i: (i, 0))


def _agg_spec():
    return pl.BlockSpec((NC, TC_BLK, 128), lambda i: (0, i, 0))


def _col_spec():
    return pl.BlockSpec((TC_BLK, 1), lambda i: (i, 0))


def _full_spec(shape):
    return pl.BlockSpec(shape, lambda i: tuple(0 for _ in shape))


def _lin0(x, W_l, W_r, b):
    return pl.pallas_call(
        _lin0_body,
        grid=(TC_GRID,),
        in_specs=[_row_spec(), _full_spec((128, 128)), _full_spec((128, 128)),
                  _full_spec((1, 128))],
        out_specs=[_row_spec(), _row_spec()],
        out_shape=[jax.ShapeDtypeStruct((N, 128), jnp.float32),
                   jax.ShapeDtypeStruct((N, 128), jnp.float32)],
    )(x, W_l, W_r, b)


def _mid(agg, inv_col, z0, W_l, W_r, b):
    return pl.pallas_call(
        _mid_body,
        grid=(TC_GRID,),
        in_specs=[_agg_spec(), _col_spec(), _row_spec(),
                  _full_spec((128, 128)), _full_spec((128, 128)),
                  _full_spec((1, 128))],
        out_specs=[_row_spec(), _row_spec()],
        out_shape=[jax.ShapeDtypeStruct((N, 128), jnp.float32),
                   jax.ShapeDtypeStruct((N, 128), jnp.float32)],
    )(agg, inv_col, z0, W_l, W_r, b)


def _fin(agg, inv_col, z1):
    return pl.pallas_call(
        _fin_body,
        grid=(TC_GRID,),
        in_specs=[_agg_spec(), _col_spec(), _row_spec()],
        out_specs=pl.BlockSpec((TC_BLK, 128), lambda i: (i, 0)),
        out_shape=jax.ShapeDtypeStruct((N, 128), jnp.float32),
    )(agg, inv_col, z1)


def kernel(x, edge_index, W_l0, b_l0, W_r0, W_l1, b_l1, W_r1):
    src_p, dst_p = _pack(edge_index)

    cnt_kernel = _make_cnt_kernel()
    agg_kernel = _make_agg_kernel()

    inv = cnt_kernel(dst_p)
    inv_col = inv[:N].reshape(N, 1)

    y0, z0 = _lin0(x, W_l0, W_r0, b_l0.reshape(1, 128))
    agg0 = agg_kernel(y0, src_p, dst_p)
    y1, z1 = _mid(agg0, inv_col, z0, W_l1, W_r1, b_l1.reshape(1, 128))
    agg1 = agg_kernel(y1, src_p, dst_p)
    return _fin(agg1, inv_col, z1)


# speedup vs baseline: 12.4004x; 1.0265x over previous
import functools

import jax
import jax.numpy as jnp
from jax import lax
from jax.experimental import pallas as pl
from jax.experimental.pallas import tpu as pltpu
from jax.experimental.pallas import tpu_sc as plsc

N = 10000
D = 128
E = 320000

NC = 2
NS = 16
NW = NC * NS

K = 64
C_PT = 160
E_PT = C_PT * K
E_PAD = NW * E_PT

N_PAD = 10240
R_BLK = N_PAD // 16


def _zero_vmem_2d(ref, rows):
    z = jnp.zeros((16,), jnp.float32)

    def body(r, _):
        for k in range(8):
            ref[r, pl.ds(k * 16, 16)] = z
        return 0

    lax.fori_loop(0, rows, body, 0)


N_CNT = N_PAD


def _zero_vmem_1d(ref, n):
    z = jnp.zeros((16,), jnp.float32)

    def body(i, _):
        ref[pl.ds(i * 16, 16)] = z
        return 0

    lax.fori_loop(0, n // 16, body, 0)


def _cnt_body(dst_hbm, cnt_hbm, dstbuf, cnt_v, acc, tmp, parts_sh):
    c = lax.axis_index("c")
    s = lax.axis_index("s")

    _zero_vmem_1d(cnt_v, N_CNT)

    ones = jnp.ones((16,), jnp.float32)
    row = c * NS + s

    def chunk(ch, _):
        pltpu.sync_copy(dst_hbm.at[row, pl.ds(ch * 1024, 1024)], dstbuf)

        def grp(g, _):
            w = dstbuf[pl.ds(g * 16, 16)]
            plsc.addupdate_scatter(cnt_v, [lax.bitwise_and(w, 0xFFFF)], ones)
            plsc.addupdate_scatter(cnt_v, [lax.shift_right_logical(w, 16)],
                                   ones)
            return 0

        lax.fori_loop(0, 64, grp, 0)
        return 0

    lax.fori_loop(0, W_PT // 1024, chunk, 0)

    pltpu.sync_copy(cnt_v, parts_sh.at[s])
    plsc.subcore_barrier()

    off = s * 640
    pltpu.sync_copy(parts_sh.at[0, pl.ds(off, 640)], acc)
    for t in range(1, NS):
        pltpu.sync_copy(parts_sh.at[t, pl.ds(off, 640)], tmp)

        def add(g, _):
            sl = pl.ds(g * 16, 16)
            acc[sl] = acc[sl] + tmp[sl]
            return 0

        lax.fori_loop(0, 40, add, 0)

    pltpu.sync_copy(acc, cnt_hbm.at[c, pl.ds(off, 640)])


def _make_cnt_kernel():
    mesh = plsc.VectorSubcoreMesh(core_axis_name="c", subcore_axis_name="s")
    return pl.kernel(
        _cnt_body,
        out_type=jax.ShapeDtypeStruct((NC, N_CNT), jnp.float32),
        mesh=mesh,
        compiler_params=pltpu.CompilerParams(needs_layout_passes=False),
        scratch_types=[
            pltpu.VMEM((1024,), jnp.int32),
            pltpu.VMEM((N_CNT,), jnp.float32),
            pltpu.VMEM((640,), jnp.float32),
            pltpu.VMEM((640,), jnp.float32),
            pltpu.VMEM_SHARED((NS, N_CNT), jnp.float32),
        ],
    )


W_PT = E_PT // 2
NBUF = 4


def _unpack_idx(packed, j, stage):
    for g in range(K // 32):
        w = packed[pl.ds(j * (K // 2) + g * 16, 16)]
        stage[0, pl.ds(g * 16, 16)] = lax.bitwise_and(w, 0xFFFF)
        stage[0, pl.ds(K // 2 + g * 16, 16)] = lax.shift_right_logical(w, 16)


def _agg_body(y_hbm, src_hbm, dst_hbm, out_hbm, src_v, dst_v, *rest):
    st_s = rest[0:NBUF]
    st_d = rest[NBUF:2 * NBUF]
    bufs = rest[2 * NBUF:3 * NBUF]
    agg_sh = rest[3 * NBUF]
    sem_g = rest[3 * NBUF + 1:3 * NBUF + 1 + NBUF]
    sem_s = rest[3 * NBUF + 1 + NBUF:]

    c = lax.axis_index("c")
    s = lax.axis_index("s")
    wid = c * NS + s

    pltpu.sync_copy(src_hbm.at[wid], src_v)
    pltpu.sync_copy(dst_hbm.at[wid], dst_v)

    _zero_vmem_2d(bufs[0], K)
    base = s * R_BLK
    nfull = (R_BLK // K) * K
    for off in range(0, nfull, K):
        pltpu.sync_copy(bufs[0], agg_sh.at[pl.ds(base + off, K)])
    if R_BLK > nfull:
        pltpu.sync_copy(bufs[0].at[pl.ds(0, R_BLK - nfull)],
                        agg_sh.at[pl.ds(base + nfull, R_BLK - nfull)])
    plsc.subcore_barrier()

    def gather_start(j, u):
        _unpack_idx(src_v, j, st_s[u])
        pltpu.async_copy(y_hbm.at[st_s[u].at[0]], bufs[u], sem_g[u])

    def wait_g(u):
        pltpu.make_async_copy(y_hbm.at[pl.ds(0, K)], bufs[u],
                              sem_g[u]).wait()

    def scat_start(j, u):
        _unpack_idx(dst_v, j, st_d[u])
        pltpu.async_copy(bufs[u], agg_sh.at[st_d[u].at[0]], sem_s[u],
                         add=True)

    def wait_s(u):
        pltpu.make_async_copy(bufs[u], agg_sh.at[pl.ds(0, K)],
                              sem_s[u]).wait()

    def slot(j, u, do_wait_s, gather_ahead):
        wait_g(u)
        scat_start(j, u)
        if gather_ahead:
            u2 = (u + 2) % NBUF
            if do_wait_s:
                wait_s(u2)
            gather_start(j + 2, u2)

    gather_start(0, 0)
    gather_start(1, 1)
    slot(0, 0, False, True)
    slot(1, 1, False, True)

    def steady(i, _):
        j = 2 + 4 * i
        slot(j, 2, True, True)
        slot(j + 1, 3, True, True)
        slot(j + 2, 0, True, True)
        slot(j + 3, 1, True, True)
        return 0

    lax.fori_loop(0, (C_PT - 8) // 4, steady, 0)

    j = C_PT - 6
    slot(j, 2, True, True)
    slot(j + 1, 3, True, True)
    slot(j + 2, 0, True, True)
    slot(j + 3, 1, True, True)
    slot(j + 4, 2, False, False)
    slot(j + 5, 3, False, False)
    for u in range(NBUF):
        wait_s(u)

    plsc.subcore_barrier()
    pltpu.sync_copy(agg_sh.at[pl.ds(base, R_BLK)],
                    out_hbm.at[c, pl.ds(base, R_BLK)])


def _make_agg_kernel():
    mesh = plsc.VectorSubcoreMesh(core_axis_name="c", subcore_axis_name="s")
    return pl.kernel(
        _agg_body,
        out_type=jax.ShapeDtypeStruct((NC, N_PAD, 128), jnp.float32),
        mesh=mesh,
        compiler_params=pltpu.CompilerParams(needs_layout_passes=False),
        scratch_types=(
            [pltpu.VMEM((W_PT,), jnp.int32)] * 2
            + [pltpu.VMEM((1, K), jnp.int32)] * (2 * NBUF)
            + [pltpu.VMEM((K, 128), jnp.float32)] * NBUF
            + [pltpu.VMEM_SHARED((N_PAD, 128), jnp.float32)]
            + [pltpu.SemaphoreType.DMA] * (2 * NBUF)
        ),
    )


TC_BLK = 2000
TC_GRID = N // TC_BLK
PACK_BL = 40960
PACK_GRID = (E_PAD // 2) // PACK_BL


def _pack_body(lo_ref, hi_ref, sp_ref, dp_ref):
    i = pl.program_id(0)
    w = ((i + PACK_GRID) * PACK_BL
         + jax.lax.broadcasted_iota(jnp.int32, (1, PACK_BL), 1))
    m = w < E
    pad_off = jnp.maximum(w - E, 0)
    hi_src = jnp.where(m, hi_ref[0:1], pad_off % N)
    hi_dst = jnp.where(m, hi_ref[1:2], N + pad_off % (N_PAD - N))
    sp_ref[...] = (lo_ref[0:1] | (hi_src << 16)).reshape(8, W_PT)
    dp_ref[...] = (lo_ref[1:2] | (hi_dst << 16)).reshape(8, W_PT)


def _pack(edge_index):
    sp, dp = pl.pallas_call(
        _pack_body,
        grid=(PACK_GRID,),
        in_specs=[pl.BlockSpec((2, PACK_BL), lambda i: (0, i)),
                  pl.BlockSpec((2, PACK_BL), lambda i: (0, i + PACK_GRID))],
        out_specs=[pl.BlockSpec((8, W_PT), lambda i: (i, 0)),
                   pl.BlockSpec((8, W_PT), lambda i: (i, 0))],
        out_shape=[jax.ShapeDtypeStruct((NW, W_PT), jnp.int32),
                   jax.ShapeDtypeStruct((NW, W_PT), jnp.int32)],
    )(edge_index, edge_index)
    return sp, dp


def _lin0_body(x_ref, wl_ref, wr_ref, b_ref, y_ref, z_ref):
    xb = x_ref[...]
    y_ref[...] = jnp.dot(xb, wl_ref[...], preferred_element_type=jnp.float32)
    z_ref[...] = (jnp.dot(xb, wr_ref[...], preferred_element_type=jnp.float32)
                  + b_ref[...])


def _mid_body(a_ref, inv_ref, z0_ref, wl_ref, wr_ref, b_ref, y_ref, z_ref):
    h = (a_ref[0] + a_ref[1]) * inv_ref[...] + z0_ref[...]
    h = jnp.maximum(h, 0.0)
    y_ref[...] = jnp.dot(h, wl_ref[...], preferred_element_type=jnp.float32)
    z_ref[...] = (jnp.dot(h, wr_ref[...], preferred_element_type=jnp.float32)
                  + b_ref[...])


def _fin_body(a_ref, inv_ref, z_ref, o_ref):
    o_ref[...] = (a_ref[0] + a_ref[1]) * inv_ref[...] + z_ref[...]


def _row_spec():
    return pl.BlockSpec((TC_BLK, 128), lambda i: (i, 0))


def _agg_spec():
    return pl.BlockSpec((NC, TC_BLK, 128), lambda i: (0, i, 0))


def _col_spec():
    return pl.BlockSpec((TC_BLK, 1), lambda i: (i, 0))


def _full_spec(shape):
    return pl.BlockSpec(shape, lambda i: tuple(0 for _ in shape))


def _lin0(x, W_l, W_r, b):
    return pl.pallas_call(
        _lin0_body,
        grid=(TC_GRID,),
        in_specs=[_row_spec(), _full_spec((128, 128)), _full_spec((128, 128)),
                  _full_spec((1, 128))],
        out_specs=[_row_spec(), _row_spec()],
        out_shape=[jax.ShapeDtypeStruct((N, 128), jnp.float32),
                   jax.ShapeDtypeStruct((N, 128), jnp.float32)],
    )(x, W_l, W_r, b)


def _mid(agg, inv_col, z0, W_l, W_r, b):
    return pl.pallas_call(
        _mid_body,
        grid=(TC_GRID,),
        in_specs=[_agg_spec(), _col_spec(), _row_spec(),
                  _full_spec((128, 128)), _full_spec((128, 128)),
                  _full_spec((1, 128))],
        out_specs=[_row_spec(), _row_spec()],
        out_shape=[jax.ShapeDtypeStruct((N, 128), jnp.float32),
                   jax.ShapeDtypeStruct((N, 128), jnp.float32)],
    )(agg, inv_col, z0, W_l, W_r, b)


def _fin(agg, inv_col, z1):
    return pl.pallas_call(
        _fin_body,
        grid=(TC_GRID,),
        in_specs=[_agg_spec(), _col_spec(), _row_spec()],
        out_specs=pl.BlockSpec((TC_BLK, 128), lambda i: (i, 0)),
        out_shape=jax.ShapeDtypeStruct((N, 128), jnp.float32),
    )(agg, inv_col, z1)


def kernel(x, edge_index, W_l0, b_l0, W_r0, W_l1, b_l1, W_r1):
    src_p, dst_p = _pack(edge_index)

    cnt_kernel = _make_cnt_kernel()
    agg_kernel = _make_agg_kernel()

    cnts = cnt_kernel(dst_p)
    inv_col = (1.0 / jnp.maximum(cnts[0] + cnts[1], 1.0))[:N].reshape(N, 1)

    y0, z0 = _lin0(x, W_l0, W_r0, b_l0.reshape(1, 128))
    agg0 = agg_kernel(y0, src_p, dst_p)
    y1, z1 = _mid(agg0, inv_col, z0, W_l1, W_r1, b_l1.reshape(1, 128))
    agg1 = agg_kernel(y1, src_p, dst_p)
    return _fin(agg1, inv_col, z1)


# speedup vs baseline: 12.5541x; 1.0124x over previous
import functools

import jax
import jax.numpy as jnp
from jax import lax
from jax.experimental import pallas as pl
from jax.experimental.pallas import tpu as pltpu
from jax.experimental.pallas import tpu_sc as plsc

N = 10000
D = 128
E = 320000

NC = 2
NS = 16
NW = NC * NS

K = 64
C_PT = 160
E_PT = C_PT * K
E_PAD = NW * E_PT

N_PAD = 10240
R_BLK = N_PAD // 16


def _zero_vmem_2d(ref, rows):
    z = jnp.zeros((16,), jnp.float32)

    def body(r, _):
        for k in range(8):
            ref[r, pl.ds(k * 16, 16)] = z
        return 0

    lax.fori_loop(0, rows, body, 0)


N_CNT = N_PAD


def _zero_vmem_1d(ref, n):
    z = jnp.zeros((16,), jnp.float32)

    def body(i, _):
        ref[pl.ds(i * 16, 16)] = z
        return 0

    lax.fori_loop(0, n // 16, body, 0)


def _cnt_body(dst_hbm, cnt_hbm, dstbuf, cnt_v, acc, tmp, parts_sh):
    c = lax.axis_index("c")
    s = lax.axis_index("s")

    _zero_vmem_1d(cnt_v, N_CNT)

    ones = jnp.ones((16,), jnp.float32)
    row = c * NS + s

    def chunk(ch, _):
        pltpu.sync_copy(dst_hbm.at[row, pl.ds(ch * 1024, 1024)], dstbuf)

        def grp(g, _):
            w = dstbuf[pl.ds(g * 16, 16)]
            plsc.addupdate_scatter(cnt_v, [lax.bitwise_and(w, 0xFFFF)], ones)
            plsc.addupdate_scatter(cnt_v, [lax.shift_right_logical(w, 16)],
                                   ones)
            return 0

        lax.fori_loop(0, 64, grp, 0)
        return 0

    lax.fori_loop(0, W_PT // 1024, chunk, 0)

    pltpu.sync_copy(cnt_v, parts_sh.at[s])
    plsc.subcore_barrier()

    off = s * 640
    pltpu.sync_copy(parts_sh.at[:, pl.ds(off, 640)], tmp)

    def add(g, _):
        sl = pl.ds(g * 16, 16)
        v = tmp[0, sl]
        for t in range(1, NS):
            v = v + tmp[t, sl]
        acc[sl] = v
        return 0

    lax.fori_loop(0, 40, add, 0)

    pltpu.sync_copy(acc, cnt_hbm.at[c, pl.ds(off, 640)])


def _make_cnt_kernel():
    mesh = plsc.VectorSubcoreMesh(core_axis_name="c", subcore_axis_name="s")
    return pl.kernel(
        _cnt_body,
        out_type=jax.ShapeDtypeStruct((NC, N_CNT), jnp.float32),
        mesh=mesh,
        compiler_params=pltpu.CompilerParams(needs_layout_passes=False),
        scratch_types=[
            pltpu.VMEM((1024,), jnp.int32),
            pltpu.VMEM((N_CNT,), jnp.float32),
            pltpu.VMEM((640,), jnp.float32),
            pltpu.VMEM((NS, 640), jnp.float32),
            pltpu.VMEM_SHARED((NS, N_CNT), jnp.float32),
        ],
    )


W_PT = E_PT // 2
NBUF = 4


def _unpack_idx(packed, j, stage):
    for g in range(K // 32):
        w = packed[pl.ds(j * (K // 2) + g * 16, 16)]
        stage[0, pl.ds(g * 16, 16)] = lax.bitwise_and(w, 0xFFFF)
        stage[0, pl.ds(K // 2 + g * 16, 16)] = lax.shift_right_logical(w, 16)


def _agg_body(y_hbm, src_hbm, dst_hbm, out_hbm, src_v, dst_v, *rest):
    st_s = rest[0:NBUF]
    st_d = rest[NBUF:2 * NBUF]
    bufs = rest[2 * NBUF:3 * NBUF]
    agg_sh = rest[3 * NBUF]
    sem_g = rest[3 * NBUF + 1:3 * NBUF + 1 + NBUF]
    sem_s = rest[3 * NBUF + 1 + NBUF:]

    c = lax.axis_index("c")
    s = lax.axis_index("s")
    wid = c * NS + s

    pltpu.sync_copy(src_hbm.at[wid], src_v)
    pltpu.sync_copy(dst_hbm.at[wid], dst_v)

    _zero_vmem_2d(bufs[0], K)
    base = s * R_BLK
    nfull = (R_BLK // K) * K
    for off in range(0, nfull, K):
        pltpu.sync_copy(bufs[0], agg_sh.at[pl.ds(base + off, K)])
    if R_BLK > nfull:
        pltpu.sync_copy(bufs[0].at[pl.ds(0, R_BLK - nfull)],
                        agg_sh.at[pl.ds(base + nfull, R_BLK - nfull)])
    plsc.subcore_barrier()

    def gather_start(j, u):
        _unpack_idx(src_v, j, st_s[u])
        pltpu.async_copy(y_hbm.at[st_s[u].at[0]], bufs[u], sem_g[u])

    def wait_g(u):
        pltpu.make_async_copy(y_hbm.at[pl.ds(0, K)], bufs[u],
                              sem_g[u]).wait()

    def scat_start(j, u):
        _unpack_idx(dst_v, j, st_d[u])
        pltpu.async_copy(bufs[u], agg_sh.at[st_d[u].at[0]], sem_s[u],
                         add=True)

    def wait_s(u):
        pltpu.make_async_copy(bufs[u], agg_sh.at[pl.ds(0, K)],
                              sem_s[u]).wait()

    def slot(j, u, do_wait_s, gather_ahead):
        wait_g(u)
        scat_start(j, u)
        if gather_ahead:
            u2 = (u + 2) % NBUF
            if do_wait_s:
                wait_s(u2)
            gather_start(j + 2, u2)

    gather_start(0, 0)
    gather_start(1, 1)
    slot(0, 0, False, True)
    slot(1, 1, False, True)

    def steady(i, _):
        j = 2 + 4 * i
        slot(j, 2, True, True)
        slot(j + 1, 3, True, True)
        slot(j + 2, 0, True, True)
        slot(j + 3, 1, True, True)
        return 0

    lax.fori_loop(0, (C_PT - 8) // 4, steady, 0)

    j = C_PT - 6
    slot(j, 2, True, True)
    slot(j + 1, 3, True, True)
    slot(j + 2, 0, True, True)
    slot(j + 3, 1, True, True)
    slot(j + 4, 2, False, False)
    slot(j + 5, 3, False, False)
    for u in range(NBUF):
        wait_s(u)

    plsc.subcore_barrier()
    pltpu.sync_copy(agg_sh.at[pl.ds(base, R_BLK)],
                    out_hbm.at[c, pl.ds(base, R_BLK)])


def _make_agg_kernel():
    mesh = plsc.VectorSubcoreMesh(core_axis_name="c", subcore_axis_name="s")
    return pl.kernel(
        _agg_body,
        out_type=jax.ShapeDtypeStruct((NC, N_PAD, 128), jnp.float32),
        mesh=mesh,
        compiler_params=pltpu.CompilerParams(needs_layout_passes=False),
        scratch_types=(
            [pltpu.VMEM((W_PT,), jnp.int32)] * 2
            + [pltpu.VMEM((1, K), jnp.int32)] * (2 * NBUF)
            + [pltpu.VMEM((K, 128), jnp.float32)] * NBUF
            + [pltpu.VMEM_SHARED((N_PAD, 128), jnp.float32)]
            + [pltpu.SemaphoreType.DMA] * (2 * NBUF)
        ),
    )


TC_BLK = 2000
TC_GRID = N // TC_BLK
PACK_BL = 40960
PACK_GRID = (E_PAD // 2) // PACK_BL


def _pack_body(lo_ref, hi_ref, sp_ref, dp_ref):
    i = pl.program_id(0)
    w = ((i + PACK_GRID) * PACK_BL
         + jax.lax.broadcasted_iota(jnp.int32, (1, PACK_BL), 1))
    m = w < E
    pad_off = jnp.maximum(w - E, 0)
    hi_src = jnp.where(m, hi_ref[0:1], pad_off % N)
    hi_dst = jnp.where(m, hi_ref[1:2], N + pad_off % (N_PAD - N))
    sp_ref[...] = (lo_ref[0:1] | (hi_src << 16)).reshape(8, W_PT)
    dp_ref[...] = (lo_ref[1:2] | (hi_dst << 16)).reshape(8, W_PT)


def _pack(edge_index):
    sp, dp = pl.pallas_call(
        _pack_body,
        grid=(PACK_GRID,),
        in_specs=[pl.BlockSpec((2, PACK_BL), lambda i: (0, i)),
                  pl.BlockSpec((2, PACK_BL), lambda i: (0, i + PACK_GRID))],
        out_specs=[pl.BlockSpec((8, W_PT), lambda i: (i, 0)),
                   pl.BlockSpec((8, W_PT), lambda i: (i, 0))],
        out_shape=[jax.ShapeDtypeStruct((NW, W_PT), jnp.int32),
                   jax.ShapeDtypeStruct((NW, W_PT), jnp.int32)],
    )(edge_index, edge_index)
    return sp, dp


def _lin0_body(x_ref, wl_ref, wr_ref, b_ref, y_ref, z_ref):
    xb = x_ref[...]
    y_ref[...] = jnp.dot(xb, wl_ref[...], preferred_element_type=jnp.float32)
    z_ref[...] = (jnp.dot(xb, wr_ref[...], preferred_element_type=jnp.float32)
                  + b_ref[...])


def _mid_body(a_ref, inv_ref, z0_ref, wl_ref, wr_ref, b_ref, y_ref, z_ref):
    h = (a_ref[0] + a_ref[1]) * inv_ref[...] + z0_ref[...]
    h = jnp.maximum(h, 0.0)
    y_ref[...] = jnp.dot(h, wl_ref[...], preferred_element_type=jnp.float32)
    z_ref[...] = (jnp.dot(h, wr_ref[...], preferred_element_type=jnp.float32)
                  + b_ref[...])


def _fin_body(a_ref, inv_ref, z_ref, o_ref):
    o_ref[...] = (a_ref[0] + a_ref[1]) * inv_ref[...] + z_ref[...]


def _row_spec():
    return pl.BlockSpec((TC_BLK, 128), lambda i: (i, 0))


def _agg_spec():
    return pl.BlockSpec((NC, TC_BLK, 128), lambda i: (0, i, 0))


def _col_spec():
    return pl.BlockSpec((TC_BLK, 1), lambda i: (i, 0))


def _full_spec(shape):
    return pl.BlockSpec(shape, lambda i: tuple(0 for _ in shape))


def _lin0(x, W_l, W_r, b):
    return pl.pallas_call(
        _lin0_body,
        grid=(TC_GRID,),
        in_specs=[_row_spec(), _full_spec((128, 128)), _full_spec((128, 128)),
                  _full_spec((1, 128))],
        out_specs=[_row_spec(), _row_spec()],
        out_shape=[jax.ShapeDtypeStruct((N, 128), jnp.float32),
                   jax.ShapeDtypeStruct((N, 128), jnp.float32)],
    )(x, W_l, W_r, b)


def _mid(agg, inv_col, z0, W_l, W_r, b):
    return pl.pallas_call(
        _mid_body,
        grid=(TC_GRID,),
        in_specs=[_agg_spec(), _col_spec(), _row_spec(),
                  _full_spec((128, 128)), _full_spec((128, 128)),
                  _full_spec((1, 128))],
        out_specs=[_row_spec(), _row_spec()],
        out_shape=[jax.ShapeDtypeStruct((N, 128), jnp.float32),
                   jax.ShapeDtypeStruct((N, 128), jnp.float32)],
    )(agg, inv_col, z0, W_l, W_r, b)


def _fin(agg, inv_col, z1):
    return pl.pallas_call(
        _fin_body,
        grid=(TC_GRID,),
        in_specs=[_agg_spec(), _col_spec(), _row_spec()],
        out_specs=pl.BlockSpec((TC_BLK, 128), lambda i: (i, 0)),
        out_shape=jax.ShapeDtypeStruct((N, 128), jnp.float32),
    )(agg, inv_col, z1)


def kernel(x, edge_index, W_l0, b_l0, W_r0, W_l1, b_l1, W_r1):
    src_p, dst_p = _pack(edge_index)

    cnt_kernel = _make_cnt_kernel()
    agg_kernel = _make_agg_kernel()

    cnts = cnt_kernel(dst_p)
    inv_col = (1.0 / jnp.maximum(cnts[0] + cnts[1], 1.0))[:N].reshape(N, 1)

    y0, z0 = _lin0(x, W_l0, W_r0, b_l0.reshape(1, 128))
    agg0 = agg_kernel(y0, src_p, dst_p)
    y1, z1 = _mid(agg0, inv_col, z0, W_l1, W_r1, b_l1.reshape(1, 128))
    agg1 = agg_kernel(y1, src_p, dst_p)
    return _fin(agg1, inv_col, z1)


# speedup vs baseline: 12.6138x; 1.0048x over previous
import functools

import jax
import jax.numpy as jnp
from jax import lax
from jax.experimental import pallas as pl
from jax.experimental.pallas import tpu as pltpu
from jax.experimental.pallas import tpu_sc as plsc

N = 10000
D = 128
E = 320000

NC = 2
NS = 16
NW = NC * NS

K = 64
C_PT = 160
E_PT = C_PT * K
E_PAD = NW * E_PT

N_PAD = 10240
R_BLK = N_PAD // 16


def _zero_vmem_2d(ref, rows):
    z = jnp.zeros((16,), jnp.float32)

    def body(r, _):
        for k in range(8):
            ref[r, pl.ds(k * 16, 16)] = z
        return 0

    lax.fori_loop(0, rows, body, 0)


N_CNT = N_PAD


def _zero_vmem_1d(ref, n):
    z = jnp.zeros((16,), jnp.float32)

    def body(i, _):
        ref[pl.ds(i * 16, 16)] = z
        return 0

    lax.fori_loop(0, n // 16, body, 0)


def _cnt_body(dst_hbm, cnt_hbm, dstbuf, cnt_v, acc, tmp, parts_sh):
    c = lax.axis_index("c")
    s = lax.axis_index("s")

    _zero_vmem_1d(cnt_v, N_CNT)

    ones = jnp.ones((16,), jnp.float32)
    row = c * NS + s

    def chunk(ch, _):
        pltpu.sync_copy(dst_hbm.at[row, pl.ds(ch * 1024, 1024)], dstbuf)

        def grp(g, _):
            w = dstbuf[pl.ds(g * 16, 16)]
            plsc.addupdate_scatter(cnt_v, [lax.bitwise_and(w, 0xFFFF)], ones)
            plsc.addupdate_scatter(cnt_v, [lax.shift_right_logical(w, 16)],
                                   ones)
            return 0

        lax.fori_loop(0, 64, grp, 0)
        return 0

    lax.fori_loop(0, W_PT // 1024, chunk, 0)

    pltpu.sync_copy(cnt_v, parts_sh.at[s])
    plsc.subcore_barrier()

    off = s * 640
    pltpu.sync_copy(parts_sh.at[:, pl.ds(off, 640)], tmp)

    def add(g, _):
        sl = pl.ds(g * 16, 16)
        v = tmp[0, sl]
        for t in range(1, NS):
            v = v + tmp[t, sl]
        acc[sl] = v
        return 0

    lax.fori_loop(0, 40, add, 0)

    pltpu.sync_copy(acc, cnt_hbm.at[c, pl.ds(off, 640)])


def _make_cnt_kernel():
    mesh = plsc.VectorSubcoreMesh(core_axis_name="c", subcore_axis_name="s")
    return pl.kernel(
        _cnt_body,
        out_type=jax.ShapeDtypeStruct((NC, N_CNT), jnp.float32),
        mesh=mesh,
        compiler_params=pltpu.CompilerParams(needs_layout_passes=False),
        scratch_types=[
            pltpu.VMEM((1024,), jnp.int32),
            pltpu.VMEM((N_CNT,), jnp.float32),
            pltpu.VMEM((640,), jnp.float32),
            pltpu.VMEM((NS, 640), jnp.float32),
            pltpu.VMEM_SHARED((NS, N_CNT), jnp.float32),
        ],
    )


W_PT = E_PT // 2
NBUF = 4


def _unpack_idx(packed, j, stage):
    for g in range(K // 32):
        w = packed[pl.ds(j * (K // 2) + g * 16, 16)]
        stage[0, pl.ds(g * 16, 16)] = lax.bitwise_and(w, 0xFFFF)
        stage[0, pl.ds(K // 2 + g * 16, 16)] = lax.shift_right_logical(w, 16)


def _agg_body(y_hbm, src_hbm, dst_hbm, out_hbm, src_v, dst_v, *rest):
    st_s = rest[0:NBUF]
    st_d = rest[NBUF:2 * NBUF]
    bufs = rest[2 * NBUF:3 * NBUF]
    agg_sh = rest[3 * NBUF]
    sem_g = rest[3 * NBUF + 1:3 * NBUF + 1 + NBUF]
    sem_s = rest[3 * NBUF + 1 + NBUF:]

    c = lax.axis_index("c")
    s = lax.axis_index("s")
    wid = c * NS + s

    pltpu.sync_copy(src_hbm.at[wid], src_v)
    pltpu.sync_copy(dst_hbm.at[wid], dst_v)

    _zero_vmem_2d(bufs[0], K)
    base = s * R_BLK
    nfull = (R_BLK // K) * K
    for off in range(0, nfull, K):
        pltpu.sync_copy(bufs[0], agg_sh.at[pl.ds(base + off, K)])
    if R_BLK > nfull:
        pltpu.sync_copy(bufs[0].at[pl.ds(0, R_BLK - nfull)],
                        agg_sh.at[pl.ds(base + nfull, R_BLK - nfull)])
    plsc.subcore_barrier()

    def gather_start(j, u):
        _unpack_idx(src_v, j, st_s[u])
        pltpu.async_copy(y_hbm.at[st_s[u].at[0]], bufs[u], sem_g[u])

    def wait_g(u):
        pltpu.make_async_copy(y_hbm.at[pl.ds(0, K)], bufs[u],
                              sem_g[u]).wait()

    def scat_start(j, u):
        _unpack_idx(dst_v, j, st_d[u])
        pltpu.async_copy(bufs[u], agg_sh.at[st_d[u].at[0]], sem_s[u],
                         add=True)

    def wait_s(u):
        pltpu.make_async_copy(bufs[u], agg_sh.at[pl.ds(0, K)],
                              sem_s[u]).wait()

    def slot(j, u, do_wait_s, gather_ahead):
        wait_g(u)
        if gather_ahead:
            u2 = (u + 2) % NBUF
            if do_wait_s:
                wait_s(u2)
            gather_start(j + 2, u2)
        scat_start(j, u)

    gather_start(0, 0)
    gather_start(1, 1)
    slot(0, 0, False, True)
    slot(1, 1, False, True)

    def steady(i, _):
        j = 2 + 4 * i
        slot(j, 2, True, True)
        slot(j + 1, 3, True, True)
        slot(j + 2, 0, True, True)
        slot(j + 3, 1, True, True)
        return 0

    lax.fori_loop(0, (C_PT - 8) // 4, steady, 0)

    j = C_PT - 6
    slot(j, 2, True, True)
    slot(j + 1, 3, True, True)
    slot(j + 2, 0, True, True)
    slot(j + 3, 1, True, True)
    slot(j + 4, 2, False, False)
    slot(j + 5, 3, False, False)
    for u in range(NBUF):
        wait_s(u)

    plsc.subcore_barrier()
    pltpu.sync_copy(agg_sh.at[pl.ds(base, R_BLK)],
                    out_hbm.at[c, pl.ds(base, R_BLK)])


def _make_agg_kernel():
    mesh = plsc.VectorSubcoreMesh(core_axis_name="c", subcore_axis_name="s")
    return pl.kernel(
        _agg_body,
        out_type=jax.ShapeDtypeStruct((NC, N_PAD, 128), jnp.float32),
        mesh=mesh,
        compiler_params=pltpu.CompilerParams(needs_layout_passes=False),
        scratch_types=(
            [pltpu.VMEM((W_PT,), jnp.int32)] * 2
            + [pltpu.VMEM((1, K), jnp.int32)] * (2 * NBUF)
            + [pltpu.VMEM((K, 128), jnp.float32)] * NBUF
            + [pltpu.VMEM_SHARED((N_PAD, 128), jnp.float32)]
            + [pltpu.SemaphoreType.DMA] * (2 * NBUF)
        ),
    )


TC_BLK = 2000
TC_GRID = N // TC_BLK
PACK_BL = 40960
PACK_GRID = (E_PAD // 2) // PACK_BL


def _pack_body(lo_ref, hi_ref, sp_ref, dp_ref):
    i = pl.program_id(0)
    w = ((i + PACK_GRID) * PACK_BL
         + jax.lax.broadcasted_iota(jnp.int32, (1, PACK_BL), 1))
    m = w < E
    pad_off = jnp.maximum(w - E, 0)
    hi_src = jnp.where(m, hi_ref[0:1], pad_off % N)
    hi_dst = jnp.where(m, hi_ref[1:2], N + pad_off % (N_PAD - N))
    sp_ref[...] = (lo_ref[0:1] | (hi_src << 16)).reshape(8, W_PT)
    dp_ref[...] = (lo_ref[1:2] | (hi_dst << 16)).reshape(8, W_PT)


def _pack(edge_index):
    sp, dp = pl.pallas_call(
        _pack_body,
        grid=(PACK_GRID,),
        in_specs=[pl.BlockSpec((2, PACK_BL), lambda i: (0, i)),
                  pl.BlockSpec((2, PACK_BL), lambda i: (0, i + PACK_GRID))],
        out_specs=[pl.BlockSpec((8, W_PT), lambda i: (i, 0)),
                   pl.BlockSpec((8, W_PT), lambda i: (i, 0))],
        out_shape=[jax.ShapeDtypeStruct((NW, W_PT), jnp.int32),
                   jax.ShapeDtypeStruct((NW, W_PT), jnp.int32)],
    )(edge_index, edge_index)
    return sp, dp


def _lin0_body(x_ref, wl_ref, wr_ref, b_ref, y_ref, z_ref):
    xb = x_ref[...]
    y_ref[...] = jnp.dot(xb, wl_ref[...], preferred_element_type=jnp.float32)
    z_ref[...] = (jnp.dot(xb, wr_ref[...], preferred_element_type=jnp.float32)
                  + b_ref[...])


def _mid_body(a_ref, inv_ref, z0_ref, wl_ref, wr_ref, b_ref, y_ref, z_ref):
    h = (a_ref[0] + a_ref[1]) * inv_ref[...] + z0_ref[...]
    h = jnp.maximum(h, 0.0)
    y_ref[...] = jnp.dot(h, wl_ref[...], preferred_element_type=jnp.float32)
    z_ref[...] = (jnp.dot(h, wr_ref[...], preferred_element_type=jnp.float32)
                  + b_ref[...])


def _fin_body(a_ref, inv_ref, z_ref, o_ref):
    o_ref[...] = (a_ref[0] + a_ref[1]) * inv_ref[...] + z_ref[...]


def _row_spec():
    return pl.BlockSpec((TC_BLK, 128), lambda i: (i, 0))


def _agg_spec():
    return pl.BlockSpec((NC, TC_BLK, 128), lambda i: (0, i, 0))


def _col_spec():
    return pl.BlockSpec((TC_BLK, 1), lambda i: (i, 0))


def _full_spec(shape):
    return pl.BlockSpec(shape, lambda i: tuple(0 for _ in shape))


def _lin0(x, W_l, W_r, b):
    return pl.pallas_call(
        _lin0_body,
        grid=(TC_GRID,),
        in_specs=[_row_spec(), _full_spec((128, 128)), _full_spec((128, 128)),
                  _full_spec((1, 128))],
        out_specs=[_row_spec(), _row_spec()],
        out_shape=[jax.ShapeDtypeStruct((N, 128), jnp.float32),
                   jax.ShapeDtypeStruct((N, 128), jnp.float32)],
    )(x, W_l, W_r, b)


def _mid(agg, inv_col, z0, W_l, W_r, b):
    return pl.pallas_call(
        _mid_body,
        grid=(TC_GRID,),
        in_specs=[_agg_spec(), _col_spec(), _row_spec(),
                  _full_spec((128, 128)), _full_spec((128, 128)),
                  _full_spec((1, 128))],
        out_specs=[_row_spec(), _row_spec()],
        out_shape=[jax.ShapeDtypeStruct((N, 128), jnp.float32),
                   jax.ShapeDtypeStruct((N, 128), jnp.float32)],
    )(agg, inv_col, z0, W_l, W_r, b)


def _fin(agg, inv_col, z1):
    return pl.pallas_call(
        _fin_body,
        grid=(TC_GRID,),
        in_specs=[_agg_spec(), _col_spec(), _row_spec()],
        out_specs=pl.BlockSpec((TC_BLK, 128), lambda i: (i, 0)),
        out_shape=jax.ShapeDtypeStruct((N, 128), jnp.float32),
    )(agg, inv_col, z1)


def kernel(x, edge_index, W_l0, b_l0, W_r0, W_l1, b_l1, W_r1):
    src_p, dst_p = _pack(edge_index)

    cnt_kernel = _make_cnt_kernel()
    agg_kernel = _make_agg_kernel()

    cnts = cnt_kernel(dst_p)
    inv_col = (1.0 / jnp.maximum(cnts[0] + cnts[1], 1.0))[:N].reshape(N, 1)

    y0, z0 = _lin0(x, W_l0, W_r0, b_l0.reshape(1, 128))
    agg0 = agg_kernel(y0, src_p, dst_p)
    y1, z1 = _mid(agg0, inv_col, z0, W_l1, W_r1, b_l1.reshape(1, 128))
    agg1 = agg_kernel(y1, src_p, dst_p)
    return _fin(agg1, inv_col, z1)
